# Initial kernel scaffold; baseline (speedup 1.0000x reference)
import jax
import jax.numpy as jnp
from jax.experimental import pallas as pl


def kernel(pc_source, pc_target, pred_flow):
    raise NotImplementedError("write your pallas kernel here")


# speedup vs baseline: 13.0973x; 13.0973x over previous
import functools

import jax
import jax.numpy as jnp
from jax.experimental import pallas as pl

RADIUS = 2.5
TQ = 128
BIG = 1e30


def _argmin_onehot(d, iota, n):
    m = jnp.min(d, axis=1, keepdims=True)
    pos = jnp.where(d == m, iota, jnp.int32(n))
    amin = jnp.min(pos, axis=1, keepdims=True)
    mask = (pos == amin).astype(jnp.float32)
    return m, mask


def _lp(x):
    return x.astype(jnp.bfloat16).astype(jnp.float32)


def _dist_block(q, ref_t):
    q0 = q[:, 0:1]
    q1 = q[:, 1:2]
    q2 = q[:, 2:3]
    r0 = ref_t[0:1, :]
    r1 = ref_t[1:2, :]
    r2 = ref_t[2:3, :]
    dot = _lp(q0) * _lp(r0) + _lp(q1) * _lp(r1) + _lp(q2) * _lp(r2)
    qq = q0 * q0 + q1 * q1 + q2 * q2
    rr = r0 * r0 + r1 * r1 + r2 * r2
    return qq + rr - 2.0 * dot


def _select_weights(d, iota, n, k):
    w = jnp.zeros(d.shape, jnp.float32)
    mask0 = None
    for t in range(k):
        m, mask = _argmin_onehot(d, iota, n)
        if t == 0:
            mask0 = mask
        keep = (m <= RADIUS).astype(jnp.float32)
        sel = mask0 + keep * (mask - mask0)
        w = w + sel
        d = d + mask * BIG
    return w


def _curv_kernel(ptq_ref, ptt_ref, psq_ref, pst_ref, warpq_ref, warpt_ref,
                 curv2_ref, curv1_ref, *, n, k):
    iota = jax.lax.broadcasted_iota(jnp.int32, (TQ, n), 1)

    q = ptq_ref[0]
    ref_t = ptt_ref[0]
    d = _dist_block(q, ref_t)
    w = _select_weights(d, iota, n, k)
    rows = []
    for c in range(3):
        s = jnp.sum(w * ref_t[c:c+1, :], axis=1)
        rows.append((s - float(k) * q[:, c]) / 9.0)
    curv2_ref[0] = jnp.stack(rows, axis=0)

    q = psq_ref[0]
    ref_t = pst_ref[0]
    wq = warpq_ref[0]
    wt = warpt_ref[0]
    d = _dist_block(q, ref_t)
    w = _select_weights(d, iota, n, k)
    rows = []
    for c in range(3):
        s = jnp.sum(w * wt[c:c+1, :], axis=1)
        rows.append((s - float(k) * wq[:, c]) / 9.0)
    curv1_ref[0] = jnp.stack(rows, axis=0)


def _interp_kernel(warpq_ref, ptt_ref, curv2t_ref, curv1t_ref, loss_ref,
                   *, n, k):
    iota = jax.lax.broadcasted_iota(jnp.int32, (TQ, n), 1)
    q = warpq_ref[0]
    ref_t = ptt_ref[0]
    d = _dist_block(q, ref_t)

    a = jnp.zeros((TQ, n), jnp.float32)
    norm = jnp.zeros((TQ, 1), jnp.float32)
    mask0 = None
    for t in range(k):
        m, mask = _argmin_onehot(d, iota, n)
        if t == 0:
            mask0 = mask
        u = 1.0 / (m + 1e-8)
        norm = norm + u
        keep = (m <= RADIUS).astype(jnp.float32)
        sel = mask0 + keep * (mask - mask0)
        a = a + u * sel
        d = d + mask * BIG

    c2 = curv2t_ref[0]
    c1 = curv1t_ref[0]
    acc = jnp.zeros((TQ,), jnp.float32)
    for c in range(3):
        inter = jnp.sum(a * c2[c:c+1, :], axis=1) / norm[:, 0]
        diff = inter - c1[c, :]
        acc = acc + diff * diff
    loss_ref[0, :, 0] = acc


@jax.jit
def kernel(pc_source, pc_target, pred_flow):
    b, n, _ = pc_source.shape
    nt = n // TQ
    warp = pc_source + pred_flow
    pt_t = jnp.transpose(pc_target, (0, 2, 1))
    ps_t = jnp.transpose(pc_source, (0, 2, 1))
    warp_t = jnp.transpose(warp, (0, 2, 1))

    q_spec = pl.BlockSpec((1, TQ, 3), lambda bi, ti: (bi, ti, 0))
    full_spec = pl.BlockSpec((1, 3, n), lambda bi, ti: (bi, 0, 0))
    out_spec = pl.BlockSpec((1, 3, TQ), lambda bi, ti: (bi, 0, ti))

    curv2_t, curv1_t = pl.pallas_call(
        functools.partial(_curv_kernel, n=n, k=10),
        grid=(b, nt),
        in_specs=[q_spec, full_spec, q_spec, full_spec, q_spec, full_spec],
        out_specs=[out_spec, out_spec],
        out_shape=[jax.ShapeDtypeStruct((b, 3, n), jnp.float32),
                   jax.ShapeDtypeStruct((b, 3, n), jnp.float32)],
    )(pc_target, pt_t, pc_source, ps_t, warp, warp_t)

    loss_terms = pl.pallas_call(
        functools.partial(_interp_kernel, n=n, k=5),
        grid=(b, nt),
        in_specs=[q_spec, full_spec, full_spec, out_spec],
        out_specs=pl.BlockSpec((1, TQ, 1), lambda bi, ti: (bi, ti, 0)),
        out_shape=jax.ShapeDtypeStruct((b, n, 1), jnp.float32),
    )(warp, pt_t, curv2_t, curv1_t)

    return jnp.sum(loss_terms) / b


# speedup vs baseline: 13.1014x; 1.0003x over previous
import functools

import jax
import jax.numpy as jnp
from jax.experimental import pallas as pl
from jax.experimental.pallas import tpu as pltpu

RADIUS = 2.5
TQ = 128
BIG = 1e30


def _argmin_onehot(d, iota, n):
    m = jnp.min(d, axis=1, keepdims=True)
    pos = jnp.where(d == m, iota, jnp.int32(n))
    amin = jnp.min(pos, axis=1, keepdims=True)
    mask = (pos == amin).astype(jnp.float32)
    return m, mask


def _lp(x):
    return x.astype(jnp.bfloat16).astype(jnp.float32)


def _dist_block(q, ref_t):
    q0 = q[:, 0:1]
    q1 = q[:, 1:2]
    q2 = q[:, 2:3]
    r0 = ref_t[0:1, :]
    r1 = ref_t[1:2, :]
    r2 = ref_t[2:3, :]
    dot = _lp(q0) * _lp(r0) + _lp(q1) * _lp(r1) + _lp(q2) * _lp(r2)
    qq = q0 * q0 + q1 * q1 + q2 * q2
    rr = r0 * r0 + r1 * r1 + r2 * r2
    return qq + rr - 2.0 * dot


def _select_weights(d, iota, n, k):
    w = jnp.zeros(d.shape, jnp.float32)
    mask0 = None
    for t in range(k):
        m, mask = _argmin_onehot(d, iota, n)
        if t == 0:
            mask0 = mask
        keep = (m <= RADIUS).astype(jnp.float32)
        sel = mask0 + keep * (mask - mask0)
        w = w + sel
        d = d + mask * BIG
    return w


def _curv_kernel(ptq_ref, ptt_ref, psq_ref, pst_ref, warpq_ref, warpt_ref,
                 curv2_ref, curv1_ref, *, n, k):
    iota = jax.lax.broadcasted_iota(jnp.int32, (TQ, n), 1)

    q = ptq_ref[0]
    ref_t = ptt_ref[0]
    d = _dist_block(q, ref_t)
    w = _select_weights(d, iota, n, k)
    rows = []
    for c in range(3):
        s = jnp.sum(w * ref_t[c:c+1, :], axis=1)
        rows.append((s - float(k) * q[:, c]) / 9.0)
    curv2_ref[0] = jnp.stack(rows, axis=0)

    q = psq_ref[0]
    ref_t = pst_ref[0]
    wq = warpq_ref[0]
    wt = warpt_ref[0]
    d = _dist_block(q, ref_t)
    w = _select_weights(d, iota, n, k)
    rows = []
    for c in range(3):
        s = jnp.sum(w * wt[c:c+1, :], axis=1)
        rows.append((s - float(k) * wq[:, c]) / 9.0)
    curv1_ref[0] = jnp.stack(rows, axis=0)


def _interp_kernel(warpq_ref, ptt_ref, curv2t_ref, curv1t_ref, loss_ref,
                   *, n, k):
    iota = jax.lax.broadcasted_iota(jnp.int32, (TQ, n), 1)
    q = warpq_ref[0]
    ref_t = ptt_ref[0]
    d = _dist_block(q, ref_t)

    a = jnp.zeros((TQ, n), jnp.float32)
    norm = jnp.zeros((TQ, 1), jnp.float32)
    mask0 = None
    for t in range(k):
        m, mask = _argmin_onehot(d, iota, n)
        if t == 0:
            mask0 = mask
        u = 1.0 / (m + 1e-8)
        norm = norm + u
        keep = (m <= RADIUS).astype(jnp.float32)
        sel = mask0 + keep * (mask - mask0)
        a = a + u * sel
        d = d + mask * BIG

    c2 = curv2t_ref[0]
    c1 = curv1t_ref[0]
    acc = jnp.zeros((TQ,), jnp.float32)
    for c in range(3):
        inter = jnp.sum(a * c2[c:c+1, :], axis=1) / norm[:, 0]
        diff = inter - c1[c, :]
        acc = acc + diff * diff
    loss_ref[0, :, 0] = acc


@jax.jit
def kernel(pc_source, pc_target, pred_flow):
    b, n, _ = pc_source.shape
    nt = n // TQ
    warp = pc_source + pred_flow
    pt_t = jnp.transpose(pc_target, (0, 2, 1))
    ps_t = jnp.transpose(pc_source, (0, 2, 1))
    warp_t = jnp.transpose(warp, (0, 2, 1))

    q_spec = pl.BlockSpec((1, TQ, 3), lambda bi, ti: (bi, ti, 0))
    full_spec = pl.BlockSpec((1, 3, n), lambda bi, ti: (bi, 0, 0))
    out_spec = pl.BlockSpec((1, 3, TQ), lambda bi, ti: (bi, 0, ti))

    curv2_t, curv1_t = pl.pallas_call(
        functools.partial(_curv_kernel, n=n, k=10),
        grid=(b, nt),
        in_specs=[q_spec, full_spec, q_spec, full_spec, q_spec, full_spec],
        out_specs=[out_spec, out_spec],
        out_shape=[jax.ShapeDtypeStruct((b, 3, n), jnp.float32),
                   jax.ShapeDtypeStruct((b, 3, n), jnp.float32)],
        compiler_params=pltpu.CompilerParams(
            dimension_semantics=("parallel", "parallel")),
    )(pc_target, pt_t, pc_source, ps_t, warp, warp_t)

    loss_terms = pl.pallas_call(
        functools.partial(_interp_kernel, n=n, k=5),
        grid=(b, nt),
        in_specs=[q_spec, full_spec, full_spec, out_spec],
        out_specs=pl.BlockSpec((1, TQ, 1), lambda bi, ti: (bi, ti, 0)),
        out_shape=jax.ShapeDtypeStruct((b, n, 1), jnp.float32),
        compiler_params=pltpu.CompilerParams(
            dimension_semantics=("parallel", "parallel")),
    )(warp, pt_t, curv2_t, curv1_t)

    return jnp.sum(loss_terms) / b


# speedup vs baseline: 15.2637x; 1.1650x over previous
import functools

import jax
import jax.numpy as jnp
from jax.experimental import pallas as pl
from jax.experimental.pallas import tpu as pltpu

RADIUS = 2.5
TQ = 128
BIG = 1e30


def _argmin_onehot(d, iota, n):
    m = jnp.min(d, axis=1, keepdims=True)
    pos = jnp.where(d == m, iota, jnp.int32(n))
    amin = jnp.min(pos, axis=1, keepdims=True)
    mask = (pos == amin).astype(jnp.float32)
    return m, mask


def _lp(x):
    return x.astype(jnp.bfloat16).astype(jnp.float32)


def _dist_block(q, ref_t):
    q0 = q[:, 0:1]
    q1 = q[:, 1:2]
    q2 = q[:, 2:3]
    r0 = ref_t[0:1, :]
    r1 = ref_t[1:2, :]
    r2 = ref_t[2:3, :]
    dot = _lp(q0) * _lp(r0) + _lp(q1) * _lp(r1) + _lp(q2) * _lp(r2)
    qq = q0 * q0 + q1 * q1 + q2 * q2
    rr = r0 * r0 + r1 * r1 + r2 * r2
    return qq + rr - 2.0 * dot


def _select_weights(d, iota, n, k):
    w = jnp.zeros(d.shape, jnp.float32)
    mask0 = None
    cnt_out = jnp.zeros((d.shape[0], 1), jnp.float32)
    for t in range(k):
        m, mask = _argmin_onehot(d, iota, n)
        if t == 0:
            mask0 = mask
        keep = (m <= RADIUS).astype(jnp.float32)
        cnt_out = cnt_out + (1.0 - keep)
        w = w + keep * mask
        if t != k - 1:
            d = d + mask * BIG
    return w + cnt_out * mask0


def _curv_kernel(ptq_ref, ptt_ref, psq_ref, pst_ref, warpq_ref, warpt_ref,
                 curv2_ref, curv1_ref, *, n, k):
    iota = jax.lax.broadcasted_iota(jnp.int32, (TQ, n), 1)

    q = ptq_ref[0]
    ref_t = ptt_ref[0]
    d = _dist_block(q, ref_t)
    w = _select_weights(d, iota, n, k)
    rows = []
    for c in range(3):
        s = jnp.sum(w * ref_t[c:c+1, :], axis=1)
        rows.append((s - float(k) * q[:, c]) / 9.0)
    curv2_ref[0] = jnp.stack(rows, axis=0)

    q = psq_ref[0]
    ref_t = pst_ref[0]
    wq = warpq_ref[0]
    wt = warpt_ref[0]
    d = _dist_block(q, ref_t)
    w = _select_weights(d, iota, n, k)
    rows = []
    for c in range(3):
        s = jnp.sum(w * wt[c:c+1, :], axis=1)
        rows.append((s - float(k) * wq[:, c]) / 9.0)
    curv1_ref[0] = jnp.stack(rows, axis=0)


def _interp_kernel(warpq_ref, ptt_ref, curv2t_ref, curv1t_ref, loss_ref,
                   *, n, k):
    iota = jax.lax.broadcasted_iota(jnp.int32, (TQ, n), 1)
    q = warpq_ref[0]
    ref_t = ptt_ref[0]
    d = _dist_block(q, ref_t)

    a = jnp.zeros((TQ, n), jnp.float32)
    norm = jnp.zeros((TQ, 1), jnp.float32)
    u_out = jnp.zeros((TQ, 1), jnp.float32)
    mask0 = None
    for t in range(k):
        m, mask = _argmin_onehot(d, iota, n)
        if t == 0:
            mask0 = mask
        u = 1.0 / (m + 1e-8)
        norm = norm + u
        keep = (m <= RADIUS).astype(jnp.float32)
        u_out = u_out + u * (1.0 - keep)
        a = a + (u * keep) * mask
        if t != k - 1:
            d = d + mask * BIG
    a = a + u_out * mask0

    c2 = curv2t_ref[0]
    c1 = curv1t_ref[0]
    acc = jnp.zeros((TQ,), jnp.float32)
    for c in range(3):
        inter = jnp.sum(a * c2[c:c+1, :], axis=1) / norm[:, 0]
        diff = inter - c1[c, :]
        acc = acc + diff * diff
    loss_ref[0, :, 0] = acc


@jax.jit
def kernel(pc_source, pc_target, pred_flow):
    b, n, _ = pc_source.shape
    nt = n // TQ
    warp = pc_source + pred_flow
    pt_t = jnp.transpose(pc_target, (0, 2, 1))
    ps_t = jnp.transpose(pc_source, (0, 2, 1))
    warp_t = jnp.transpose(warp, (0, 2, 1))

    q_spec = pl.BlockSpec((1, TQ, 3), lambda bi, ti: (bi, ti, 0))
    full_spec = pl.BlockSpec((1, 3, n), lambda bi, ti: (bi, 0, 0))
    out_spec = pl.BlockSpec((1, 3, TQ), lambda bi, ti: (bi, 0, ti))

    curv2_t, curv1_t = pl.pallas_call(
        functools.partial(_curv_kernel, n=n, k=10),
        grid=(b, nt),
        in_specs=[q_spec, full_spec, q_spec, full_spec, q_spec, full_spec],
        out_specs=[out_spec, out_spec],
        out_shape=[jax.ShapeDtypeStruct((b, 3, n), jnp.float32),
                   jax.ShapeDtypeStruct((b, 3, n), jnp.float32)],
        compiler_params=pltpu.CompilerParams(
            dimension_semantics=("parallel", "parallel")),
    )(pc_target, pt_t, pc_source, ps_t, warp, warp_t)

    loss_terms = pl.pallas_call(
        functools.partial(_interp_kernel, n=n, k=5),
        grid=(b, nt),
        in_specs=[q_spec, full_spec, full_spec, out_spec],
        out_specs=pl.BlockSpec((1, TQ, 1), lambda bi, ti: (bi, ti, 0)),
        out_shape=jax.ShapeDtypeStruct((b, n, 1), jnp.float32),
        compiler_params=pltpu.CompilerParams(
            dimension_semantics=("parallel", "parallel")),
    )(warp, pt_t, curv2_t, curv1_t)

    return jnp.sum(loss_terms) / b


# speedup vs baseline: 18.0242x; 1.1809x over previous
import functools

import jax
import jax.numpy as jnp
from jax.experimental import pallas as pl
from jax.experimental.pallas import tpu as pltpu

RADIUS = 2.5
TQ = 128
BIG = 1e30


def _argmin_mask(d):
    m = jnp.min(d, axis=1, keepdims=True)
    mask = (d == m).astype(jnp.float32)
    inv_c = 1.0 / jnp.sum(mask, axis=1, keepdims=True)
    return m, mask, inv_c


def _lp(x):
    return x.astype(jnp.bfloat16).astype(jnp.float32)


def _dist_block(q, ref_t):
    q0 = q[:, 0:1]
    q1 = q[:, 1:2]
    q2 = q[:, 2:3]
    r0 = ref_t[0:1, :]
    r1 = ref_t[1:2, :]
    r2 = ref_t[2:3, :]
    dot = _lp(q0) * _lp(r0) + _lp(q1) * _lp(r1) + _lp(q2) * _lp(r2)
    qq = q0 * q0 + q1 * q1 + q2 * q2
    rr = r0 * r0 + r1 * r1 + r2 * r2
    return qq + rr - 2.0 * dot


def _select_weights(d, k):
    w = jnp.zeros(d.shape, jnp.float32)
    mask0 = inv_c0 = None
    cnt_out = jnp.zeros((d.shape[0], 1), jnp.float32)
    for t in range(k):
        m, mask, inv_c = _argmin_mask(d)
        if t == 0:
            mask0, inv_c0 = mask, inv_c
        keep = (m <= RADIUS).astype(jnp.float32)
        cnt_out = cnt_out + (1.0 - keep)
        w = w + (keep * inv_c) * mask
        if t != k - 1:
            d = d + mask * BIG
    return w + (cnt_out * inv_c0) * mask0


def _curv_kernel(ptq_ref, ptt_ref, psq_ref, pst_ref, warpq_ref, warpt_ref,
                 curv2_ref, curv1_ref, *, n, k):

    q = ptq_ref[0]
    ref_t = ptt_ref[0]
    d = _dist_block(q, ref_t)
    w = _select_weights(d, k)
    rows = []
    for c in range(3):
        s = jnp.sum(w * ref_t[c:c+1, :], axis=1)
        rows.append((s - float(k) * q[:, c]) / 9.0)
    curv2_ref[0] = jnp.stack(rows, axis=0)

    q = psq_ref[0]
    ref_t = pst_ref[0]
    wq = warpq_ref[0]
    wt = warpt_ref[0]
    d = _dist_block(q, ref_t)
    w = _select_weights(d, k)
    rows = []
    for c in range(3):
        s = jnp.sum(w * wt[c:c+1, :], axis=1)
        rows.append((s - float(k) * wq[:, c]) / 9.0)
    curv1_ref[0] = jnp.stack(rows, axis=0)


def _interp_kernel(warpq_ref, ptt_ref, curv2t_ref, curv1t_ref, loss_ref,
                   *, n, k):
    q = warpq_ref[0]
    ref_t = ptt_ref[0]
    d = _dist_block(q, ref_t)

    a = jnp.zeros((TQ, n), jnp.float32)
    norm = jnp.zeros((TQ, 1), jnp.float32)
    u_out = jnp.zeros((TQ, 1), jnp.float32)
    mask0 = inv_c0 = None
    for t in range(k):
        m, mask, inv_c = _argmin_mask(d)
        if t == 0:
            mask0, inv_c0 = mask, inv_c
        u = 1.0 / (m + 1e-8)
        norm = norm + u
        keep = (m <= RADIUS).astype(jnp.float32)
        u_out = u_out + u * (1.0 - keep)
        a = a + (u * keep * inv_c) * mask
        if t != k - 1:
            d = d + mask * BIG
    a = a + (u_out * inv_c0) * mask0

    c2 = curv2t_ref[0]
    c1 = curv1t_ref[0]
    acc = jnp.zeros((TQ,), jnp.float32)
    for c in range(3):
        inter = jnp.sum(a * c2[c:c+1, :], axis=1) / norm[:, 0]
        diff = inter - c1[c, :]
        acc = acc + diff * diff
    loss_ref[0, :, 0] = acc


@jax.jit
def kernel(pc_source, pc_target, pred_flow):
    b, n, _ = pc_source.shape
    nt = n // TQ
    warp = pc_source + pred_flow
    pt_t = jnp.transpose(pc_target, (0, 2, 1))
    ps_t = jnp.transpose(pc_source, (0, 2, 1))
    warp_t = jnp.transpose(warp, (0, 2, 1))

    q_spec = pl.BlockSpec((1, TQ, 3), lambda bi, ti: (bi, ti, 0))
    full_spec = pl.BlockSpec((1, 3, n), lambda bi, ti: (bi, 0, 0))
    out_spec = pl.BlockSpec((1, 3, TQ), lambda bi, ti: (bi, 0, ti))

    curv2_t, curv1_t = pl.pallas_call(
        functools.partial(_curv_kernel, n=n, k=10),
        grid=(b, nt),
        in_specs=[q_spec, full_spec, q_spec, full_spec, q_spec, full_spec],
        out_specs=[out_spec, out_spec],
        out_shape=[jax.ShapeDtypeStruct((b, 3, n), jnp.float32),
                   jax.ShapeDtypeStruct((b, 3, n), jnp.float32)],
        compiler_params=pltpu.CompilerParams(
            dimension_semantics=("parallel", "parallel")),
    )(pc_target, pt_t, pc_source, ps_t, warp, warp_t)

    loss_terms = pl.pallas_call(
        functools.partial(_interp_kernel, n=n, k=5),
        grid=(b, nt),
        in_specs=[q_spec, full_spec, full_spec, out_spec],
        out_specs=pl.BlockSpec((1, TQ, 1), lambda bi, ti: (bi, ti, 0)),
        out_shape=jax.ShapeDtypeStruct((b, n, 1), jnp.float32),
        compiler_params=pltpu.CompilerParams(
            dimension_semantics=("parallel", "parallel")),
    )(warp, pt_t, curv2_t, curv1_t)

    return jnp.sum(loss_terms) / b


# speedup vs baseline: 18.1648x; 1.0078x over previous
import functools

import jax
import jax.numpy as jnp
from jax.experimental import pallas as pl
from jax.experimental.pallas import tpu as pltpu

RADIUS = 2.5
TQ = 256
BIG = 1e30


def _argmin_mask(d):
    m = jnp.min(d, axis=1, keepdims=True)
    mask = (d == m).astype(jnp.float32)
    inv_c = 1.0 / jnp.sum(mask, axis=1, keepdims=True)
    return m, mask, inv_c


def _lp(x):
    return x.astype(jnp.bfloat16).astype(jnp.float32)


def _dist_block(q, ref_t):
    q0 = q[:, 0:1]
    q1 = q[:, 1:2]
    q2 = q[:, 2:3]
    r0 = ref_t[0:1, :]
    r1 = ref_t[1:2, :]
    r2 = ref_t[2:3, :]
    dot = _lp(q0) * _lp(r0) + _lp(q1) * _lp(r1) + _lp(q2) * _lp(r2)
    qq = q0 * q0 + q1 * q1 + q2 * q2
    rr = r0 * r0 + r1 * r1 + r2 * r2
    return qq + rr - 2.0 * dot


def _select_weights(d, k):
    w = jnp.zeros(d.shape, jnp.float32)
    mask0 = inv_c0 = None
    cnt_out = jnp.zeros((d.shape[0], 1), jnp.float32)
    for t in range(k):
        m, mask, inv_c = _argmin_mask(d)
        if t == 0:
            mask0, inv_c0 = mask, inv_c
        keep = (m <= RADIUS).astype(jnp.float32)
        cnt_out = cnt_out + (1.0 - keep)
        w = w + (keep * inv_c) * mask
        if t != k - 1:
            d = d + mask * BIG
    return w + (cnt_out * inv_c0) * mask0


def _curv_kernel(ptq_ref, ptt_ref, psq_ref, pst_ref, warpq_ref, warpt_ref,
                 curv2_ref, curv1_ref, *, n, k):

    q = ptq_ref[0]
    ref_t = ptt_ref[0]
    d = _dist_block(q, ref_t)
    w = _select_weights(d, k)
    rows = []
    for c in range(3):
        s = jnp.sum(w * ref_t[c:c+1, :], axis=1)
        rows.append((s - float(k) * q[:, c]) / 9.0)
    curv2_ref[0] = jnp.stack(rows, axis=0)

    q = psq_ref[0]
    ref_t = pst_ref[0]
    wq = warpq_ref[0]
    wt = warpt_ref[0]
    d = _dist_block(q, ref_t)
    w = _select_weights(d, k)
    rows = []
    for c in range(3):
        s = jnp.sum(w * wt[c:c+1, :], axis=1)
        rows.append((s - float(k) * wq[:, c]) / 9.0)
    curv1_ref[0] = jnp.stack(rows, axis=0)


def _interp_kernel(warpq_ref, ptt_ref, curv2t_ref, curv1t_ref, loss_ref,
                   *, n, k):
    q = warpq_ref[0]
    ref_t = ptt_ref[0]
    d = _dist_block(q, ref_t)

    a = jnp.zeros((TQ, n), jnp.float32)
    norm = jnp.zeros((TQ, 1), jnp.float32)
    u_out = jnp.zeros((TQ, 1), jnp.float32)
    mask0 = inv_c0 = None
    for t in range(k):
        m, mask, inv_c = _argmin_mask(d)
        if t == 0:
            mask0, inv_c0 = mask, inv_c
        u = 1.0 / (m + 1e-8)
        norm = norm + u
        keep = (m <= RADIUS).astype(jnp.float32)
        u_out = u_out + u * (1.0 - keep)
        a = a + (u * keep * inv_c) * mask
        if t != k - 1:
            d = d + mask * BIG
    a = a + (u_out * inv_c0) * mask0

    c2 = curv2t_ref[0]
    c1 = curv1t_ref[0]
    acc = jnp.zeros((TQ,), jnp.float32)
    for c in range(3):
        inter = jnp.sum(a * c2[c:c+1, :], axis=1) / norm[:, 0]
        diff = inter - c1[c, :]
        acc = acc + diff * diff
    loss_ref[0, :, 0] = acc


@jax.jit
def kernel(pc_source, pc_target, pred_flow):
    b, n, _ = pc_source.shape
    nt = n // TQ
    warp = pc_source + pred_flow
    pt_t = jnp.transpose(pc_target, (0, 2, 1))
    ps_t = jnp.transpose(pc_source, (0, 2, 1))
    warp_t = jnp.transpose(warp, (0, 2, 1))

    q_spec = pl.BlockSpec((1, TQ, 3), lambda bi, ti: (bi, ti, 0))
    full_spec = pl.BlockSpec((1, 3, n), lambda bi, ti: (bi, 0, 0))
    out_spec = pl.BlockSpec((1, 3, TQ), lambda bi, ti: (bi, 0, ti))

    curv2_t, curv1_t = pl.pallas_call(
        functools.partial(_curv_kernel, n=n, k=10),
        grid=(b, nt),
        in_specs=[q_spec, full_spec, q_spec, full_spec, q_spec, full_spec],
        out_specs=[out_spec, out_spec],
        out_shape=[jax.ShapeDtypeStruct((b, 3, n), jnp.float32),
                   jax.ShapeDtypeStruct((b, 3, n), jnp.float32)],
        compiler_params=pltpu.CompilerParams(
            dimension_semantics=("parallel", "parallel")),
    )(pc_target, pt_t, pc_source, ps_t, warp, warp_t)

    loss_terms = pl.pallas_call(
        functools.partial(_interp_kernel, n=n, k=5),
        grid=(b, nt),
        in_specs=[q_spec, full_spec, full_spec, out_spec],
        out_specs=pl.BlockSpec((1, TQ, 1), lambda bi, ti: (bi, ti, 0)),
        out_shape=jax.ShapeDtypeStruct((b, n, 1), jnp.float32),
        compiler_params=pltpu.CompilerParams(
            dimension_semantics=("parallel", "parallel")),
    )(warp, pt_t, curv2_t, curv1_t)

    return jnp.sum(loss_terms) / b


# speedup vs baseline: 34.8109x; 1.9164x over previous
import functools

import jax
import jax.numpy as jnp
from jax.experimental import pallas as pl
from jax.experimental.pallas import tpu as pltpu

RADIUS = 2.5
TQ = 256
BIG = 1e30


def _kth_threshold(d, k):
    m0 = None
    for t in range(k - 1):
        m = jnp.min(d, axis=1, keepdims=True)
        if t == 0:
            m0 = m
        d = jnp.where(d == m, BIG, d)
    theta = jnp.min(d, axis=1, keepdims=True)
    if m0 is None:
        m0 = theta
    return m0, theta


def _lp(x):
    return x.astype(jnp.bfloat16).astype(jnp.float32)


def _dist_block(q, ref_t):
    q0 = q[:, 0:1]
    q1 = q[:, 1:2]
    q2 = q[:, 2:3]
    r0 = ref_t[0:1, :]
    r1 = ref_t[1:2, :]
    r2 = ref_t[2:3, :]
    dot = _lp(q0) * _lp(r0) + _lp(q1) * _lp(r1) + _lp(q2) * _lp(r2)
    qq = q0 * q0 + q1 * q1 + q2 * q2
    rr = r0 * r0 + r1 * r1 + r2 * r2
    return qq + rr - 2.0 * dot


def _select_weights(d, k):
    m0, theta = _kth_threshold(d, k)
    sel = (d <= theta).astype(jnp.float32)
    w_in = sel * (d <= RADIUS).astype(jnp.float32)
    cnt_out = float(k) - jnp.sum(w_in, axis=1, keepdims=True)
    mask0 = (d == m0).astype(jnp.float32)
    inv_c0 = 1.0 / jnp.sum(mask0, axis=1, keepdims=True)
    return w_in + (cnt_out * inv_c0) * mask0


def _curv_kernel(ptq_ref, ptt_ref, psq_ref, pst_ref, warpq_ref, warpt_ref,
                 curv2_ref, curv1_ref, *, n, k):

    q = ptq_ref[0]
    ref_t = ptt_ref[0]
    d = _dist_block(q, ref_t)
    w = _select_weights(d, k)
    rows = []
    for c in range(3):
        s = jnp.sum(w * ref_t[c:c+1, :], axis=1)
        rows.append((s - float(k) * q[:, c]) / 9.0)
    curv2_ref[0] = jnp.stack(rows, axis=0)

    q = psq_ref[0]
    ref_t = pst_ref[0]
    wq = warpq_ref[0]
    wt = warpt_ref[0]
    d = _dist_block(q, ref_t)
    w = _select_weights(d, k)
    rows = []
    for c in range(3):
        s = jnp.sum(w * wt[c:c+1, :], axis=1)
        rows.append((s - float(k) * wq[:, c]) / 9.0)
    curv1_ref[0] = jnp.stack(rows, axis=0)


def _interp_kernel(warpq_ref, ptt_ref, curv2t_ref, curv1t_ref, loss_ref,
                   *, n, k):
    q = warpq_ref[0]
    ref_t = ptt_ref[0]
    d = _dist_block(q, ref_t)

    m0, theta = _kth_threshold(d, k)
    sel = d <= theta
    uv = jnp.where(sel, 1.0 / (d + 1e-8), 0.0)
    norm = jnp.sum(uv, axis=1, keepdims=True)
    keepv = (d <= RADIUS).astype(jnp.float32)
    a_in = uv * keepv
    u_out = jnp.sum(uv * (1.0 - keepv), axis=1, keepdims=True)
    mask0 = (d == m0).astype(jnp.float32)
    inv_c0 = 1.0 / jnp.sum(mask0, axis=1, keepdims=True)
    a = a_in + (u_out * inv_c0) * mask0

    c2 = curv2t_ref[0]
    c1 = curv1t_ref[0]
    acc = jnp.zeros((TQ,), jnp.float32)
    for c in range(3):
        inter = jnp.sum(a * c2[c:c+1, :], axis=1) / norm[:, 0]
        diff = inter - c1[c, :]
        acc = acc + diff * diff
    loss_ref[0, :, 0] = acc


@jax.jit
def kernel(pc_source, pc_target, pred_flow):
    b, n, _ = pc_source.shape
    nt = n // TQ
    warp = pc_source + pred_flow
    pt_t = jnp.transpose(pc_target, (0, 2, 1))
    ps_t = jnp.transpose(pc_source, (0, 2, 1))
    warp_t = jnp.transpose(warp, (0, 2, 1))

    q_spec = pl.BlockSpec((1, TQ, 3), lambda bi, ti: (bi, ti, 0))
    full_spec = pl.BlockSpec((1, 3, n), lambda bi, ti: (bi, 0, 0))
    out_spec = pl.BlockSpec((1, 3, TQ), lambda bi, ti: (bi, 0, ti))

    curv2_t, curv1_t = pl.pallas_call(
        functools.partial(_curv_kernel, n=n, k=10),
        grid=(b, nt),
        in_specs=[q_spec, full_spec, q_spec, full_spec, q_spec, full_spec],
        out_specs=[out_spec, out_spec],
        out_shape=[jax.ShapeDtypeStruct((b, 3, n), jnp.float32),
                   jax.ShapeDtypeStruct((b, 3, n), jnp.float32)],
        compiler_params=pltpu.CompilerParams(
            dimension_semantics=("parallel", "parallel")),
    )(pc_target, pt_t, pc_source, ps_t, warp, warp_t)

    loss_terms = pl.pallas_call(
        functools.partial(_interp_kernel, n=n, k=5),
        grid=(b, nt),
        in_specs=[q_spec, full_spec, full_spec, out_spec],
        out_specs=pl.BlockSpec((1, TQ, 1), lambda bi, ti: (bi, ti, 0)),
        out_shape=jax.ShapeDtypeStruct((b, n, 1), jnp.float32),
        compiler_params=pltpu.CompilerParams(
            dimension_semantics=("parallel", "parallel")),
    )(warp, pt_t, curv2_t, curv1_t)

    return jnp.sum(loss_terms) / b


# speedup vs baseline: 35.3410x; 1.0152x over previous
import functools

import jax
import jax.numpy as jnp
from jax.experimental import pallas as pl
from jax.experimental.pallas import tpu as pltpu

RADIUS = 2.5
TQ = 256
BIG = 1e30


def _kth_threshold(d, k):
    m = jnp.min(d, axis=1, keepdims=True)
    m0 = m
    for _ in range(k - 1):
        m = jnp.min(jnp.where(d > m, d, BIG), axis=1, keepdims=True)
    return m0, m


def _lp(x):
    return x.astype(jnp.bfloat16).astype(jnp.float32)


def _dist_block(q, ref_t):
    q0 = q[:, 0:1]
    q1 = q[:, 1:2]
    q2 = q[:, 2:3]
    r0 = ref_t[0:1, :]
    r1 = ref_t[1:2, :]
    r2 = ref_t[2:3, :]
    dot = _lp(q0) * _lp(r0) + _lp(q1) * _lp(r1) + _lp(q2) * _lp(r2)
    qq = q0 * q0 + q1 * q1 + q2 * q2
    rr = r0 * r0 + r1 * r1 + r2 * r2
    return qq + rr - 2.0 * dot


def _select_weights(d, k):
    m0, theta = _kth_threshold(d, k)
    sel = (d <= theta).astype(jnp.float32)
    w_in = sel * (d <= RADIUS).astype(jnp.float32)
    cnt_out = float(k) - jnp.sum(w_in, axis=1, keepdims=True)
    mask0 = (d == m0).astype(jnp.float32)
    inv_c0 = 1.0 / jnp.sum(mask0, axis=1, keepdims=True)
    return w_in + (cnt_out * inv_c0) * mask0


def _curv_kernel(ptq_ref, ptt_ref, psq_ref, pst_ref, warpq_ref, warpt_ref,
                 curv2_ref, curv1_ref, *, n, k):

    q = ptq_ref[0]
    ref_t = ptt_ref[0]
    d = _dist_block(q, ref_t)
    w = _select_weights(d, k)
    rows = []
    for c in range(3):
        s = jnp.sum(w * ref_t[c:c+1, :], axis=1)
        rows.append((s - float(k) * q[:, c]) / 9.0)
    curv2_ref[0] = jnp.stack(rows, axis=0)

    q = psq_ref[0]
    ref_t = pst_ref[0]
    wq = warpq_ref[0]
    wt = warpt_ref[0]
    d = _dist_block(q, ref_t)
    w = _select_weights(d, k)
    rows = []
    for c in range(3):
        s = jnp.sum(w * wt[c:c+1, :], axis=1)
        rows.append((s - float(k) * wq[:, c]) / 9.0)
    curv1_ref[0] = jnp.stack(rows, axis=0)


def _interp_kernel(warpq_ref, ptt_ref, curv2t_ref, curv1t_ref, loss_ref,
                   *, n, k):
    q = warpq_ref[0]
    ref_t = ptt_ref[0]
    d = _dist_block(q, ref_t)

    m0, theta = _kth_threshold(d, k)
    sel = d <= theta
    uv = jnp.where(sel, 1.0 / (d + 1e-8), 0.0)
    norm = jnp.sum(uv, axis=1, keepdims=True)
    keepv = (d <= RADIUS).astype(jnp.float32)
    a_in = uv * keepv
    u_out = jnp.sum(uv * (1.0 - keepv), axis=1, keepdims=True)
    mask0 = (d == m0).astype(jnp.float32)
    inv_c0 = 1.0 / jnp.sum(mask0, axis=1, keepdims=True)
    a = a_in + (u_out * inv_c0) * mask0

    c2 = curv2t_ref[0]
    c1 = curv1t_ref[0]
    acc = jnp.zeros((TQ,), jnp.float32)
    for c in range(3):
        inter = jnp.sum(a * c2[c:c+1, :], axis=1) / norm[:, 0]
        diff = inter - c1[c, :]
        acc = acc + diff * diff
    loss_ref[0, :, 0] = acc


@jax.jit
def kernel(pc_source, pc_target, pred_flow):
    b, n, _ = pc_source.shape
    nt = n // TQ
    warp = pc_source + pred_flow
    pt_t = jnp.transpose(pc_target, (0, 2, 1))
    ps_t = jnp.transpose(pc_source, (0, 2, 1))
    warp_t = jnp.transpose(warp, (0, 2, 1))

    q_spec = pl.BlockSpec((1, TQ, 3), lambda bi, ti: (bi, ti, 0))
    full_spec = pl.BlockSpec((1, 3, n), lambda bi, ti: (bi, 0, 0))
    out_spec = pl.BlockSpec((1, 3, TQ), lambda bi, ti: (bi, 0, ti))

    curv2_t, curv1_t = pl.pallas_call(
        functools.partial(_curv_kernel, n=n, k=10),
        grid=(b, nt),
        in_specs=[q_spec, full_spec, q_spec, full_spec, q_spec, full_spec],
        out_specs=[out_spec, out_spec],
        out_shape=[jax.ShapeDtypeStruct((b, 3, n), jnp.float32),
                   jax.ShapeDtypeStruct((b, 3, n), jnp.float32)],
        compiler_params=pltpu.CompilerParams(
            dimension_semantics=("parallel", "parallel")),
    )(pc_target, pt_t, pc_source, ps_t, warp, warp_t)

    loss_terms = pl.pallas_call(
        functools.partial(_interp_kernel, n=n, k=5),
        grid=(b, nt),
        in_specs=[q_spec, full_spec, full_spec, out_spec],
        out_specs=pl.BlockSpec((1, TQ, 1), lambda bi, ti: (bi, ti, 0)),
        out_shape=jax.ShapeDtypeStruct((b, n, 1), jnp.float32),
        compiler_params=pltpu.CompilerParams(
            dimension_semantics=("parallel", "parallel")),
    )(warp, pt_t, curv2_t, curv1_t)

    return jnp.sum(loss_terms) / b


# speedup vs baseline: 36.4615x; 1.0317x over previous
import functools

import jax
import jax.numpy as jnp
from jax.experimental import pallas as pl
from jax.experimental.pallas import tpu as pltpu

RADIUS = 2.5
TQ = 256
BIG = 1e30


def _kth_threshold(d, k):
    m = jnp.min(d, axis=1, keepdims=True)
    m0 = m
    for _ in range(k - 1):
        m = jnp.min(jnp.where(d > m, d, BIG), axis=1, keepdims=True)
    return m0, m


def _lp(x):
    return x.astype(jnp.bfloat16).astype(jnp.float32)


def _dist_block(q, ref_t):
    q0 = q[:, 0:1]
    q1 = q[:, 1:2]
    q2 = q[:, 2:3]
    r0 = ref_t[0:1, :]
    r1 = ref_t[1:2, :]
    r2 = ref_t[2:3, :]
    dot = _lp(q0) * _lp(r0) + _lp(q1) * _lp(r1) + _lp(q2) * _lp(r2)
    qq = q0 * q0 + q1 * q1 + q2 * q2
    rr = r0 * r0 + r1 * r1 + r2 * r2
    return qq + rr - 2.0 * dot


def _select_weights(d, k):
    m0, theta = _kth_threshold(d, k)
    w_in = (d <= jnp.minimum(theta, RADIUS)).astype(jnp.float32)
    cnt_out = float(k) - jnp.sum(w_in, axis=1, keepdims=True)
    mask0 = (d == m0).astype(jnp.float32)
    inv_c0 = 1.0 / jnp.sum(mask0, axis=1, keepdims=True)
    return w_in + (cnt_out * inv_c0) * mask0


def _curv_kernel(ptq_ref, ptt_ref, psq_ref, pst_ref, warpq_ref, warpt_ref,
                 curv2_ref, curv1_ref, *, n, k):

    q = ptq_ref[0]
    ref_t = ptt_ref[0]
    d = _dist_block(q, ref_t)
    w = _select_weights(d, k)
    rows = []
    for c in range(3):
        s = jnp.sum(w * ref_t[c:c+1, :], axis=1)
        rows.append((s - float(k) * q[:, c]) / 9.0)
    curv2_ref[0] = jnp.stack(rows, axis=0)

    q = psq_ref[0]
    ref_t = pst_ref[0]
    wq = warpq_ref[0]
    wt = warpt_ref[0]
    d = _dist_block(q, ref_t)
    w = _select_weights(d, k)
    rows = []
    for c in range(3):
        s = jnp.sum(w * wt[c:c+1, :], axis=1)
        rows.append((s - float(k) * wq[:, c]) / 9.0)
    curv1_ref[0] = jnp.stack(rows, axis=0)


def _interp_kernel(warpq_ref, ptt_ref, curv2t_ref, curv1t_ref, loss_ref,
                   *, n, k):
    q = warpq_ref[0]
    ref_t = ptt_ref[0]
    d = _dist_block(q, ref_t)

    m0, theta = _kth_threshold(d, k)
    sel = d <= theta
    uv = jnp.where(sel, 1.0 / (d + 1e-8), 0.0)
    norm = jnp.sum(uv, axis=1, keepdims=True)
    keepv = (d <= RADIUS).astype(jnp.float32)
    a_in = uv * keepv
    u_out = jnp.sum(uv * (1.0 - keepv), axis=1, keepdims=True)
    mask0 = (d == m0).astype(jnp.float32)
    inv_c0 = 1.0 / jnp.sum(mask0, axis=1, keepdims=True)
    a = a_in + (u_out * inv_c0) * mask0

    c2 = curv2t_ref[0]
    c1 = curv1t_ref[0]
    acc = jnp.zeros((TQ,), jnp.float32)
    for c in range(3):
        inter = jnp.sum(a * c2[c:c+1, :], axis=1) / norm[:, 0]
        diff = inter - c1[c, :]
        acc = acc + diff * diff
    loss_ref[0, :, 0] = acc


@jax.jit
def kernel(pc_source, pc_target, pred_flow):
    b, n, _ = pc_source.shape
    nt = n // TQ
    warp = pc_source + pred_flow
    pt_t = jnp.transpose(pc_target, (0, 2, 1))
    ps_t = jnp.transpose(pc_source, (0, 2, 1))
    warp_t = jnp.transpose(warp, (0, 2, 1))

    q_spec = pl.BlockSpec((1, TQ, 3), lambda bi, ti: (bi, ti, 0))
    full_spec = pl.BlockSpec((1, 3, n), lambda bi, ti: (bi, 0, 0))
    out_spec = pl.BlockSpec((1, 3, TQ), lambda bi, ti: (bi, 0, ti))

    curv2_t, curv1_t = pl.pallas_call(
        functools.partial(_curv_kernel, n=n, k=10),
        grid=(b, nt),
        in_specs=[q_spec, full_spec, q_spec, full_spec, q_spec, full_spec],
        out_specs=[out_spec, out_spec],
        out_shape=[jax.ShapeDtypeStruct((b, 3, n), jnp.float32),
                   jax.ShapeDtypeStruct((b, 3, n), jnp.float32)],
        compiler_params=pltpu.CompilerParams(
            dimension_semantics=("parallel", "parallel")),
    )(pc_target, pt_t, pc_source, ps_t, warp, warp_t)

    loss_terms = pl.pallas_call(
        functools.partial(_interp_kernel, n=n, k=5),
        grid=(b, nt),
        in_specs=[q_spec, full_spec, full_spec, out_spec],
        out_specs=pl.BlockSpec((1, TQ, 1), lambda bi, ti: (bi, ti, 0)),
        out_shape=jax.ShapeDtypeStruct((b, n, 1), jnp.float32),
        compiler_params=pltpu.CompilerParams(
            dimension_semantics=("parallel", "parallel")),
    )(warp, pt_t, curv2_t, curv1_t)

    return jnp.sum(loss_terms) / b
